# baseline (speedup 1.0000x reference)
import functools

import jax
import jax.numpy as jnp
from jax import lax
from jax.experimental import pallas as pl
from jax.experimental.pallas import tpu as pltpu
from jax.experimental.pallas import tpu_sc as plsc

_N = 50000
_E = 800000
_D = 64
_NC = 2
_NS = 16
_NPASS = 4
_W = 128
_K = 2
_CH = 392
_EPAD = _NS * _W * _CH
_NPS = 3128
_NP = _NS * _NPS
_BD = _D // (_NC * _NPASS)
_LAST = _N - (_NS - 1) * _NPS
_T = _CH // _K


def _sc_body(f8_hbm, ei_hbm, z_hbm, out_hbm, srcv, dstv, stage_a, stage_b,
             rows_a, rows_b, acc, gs_a, gs_b, ss_a, ss_b):
    c = lax.axis_index("c")
    s = lax.axis_index("s")
    base = s * _NPS

    pltpu.sync_copy(ei_hbm.at[0, s], srcv)
    pltpu.sync_copy(ei_hbm.at[1, s], dstv)

    def fire_g(t, stage, rbuf, sem, bvec):
        for i in range(_K):
            for k in range(8):
                sl = pl.ds(k * 16, 16)
                stage[i, sl] = srcv[t * _K + i, sl] * 8 + bvec
        for i in range(_K):
            pltpu.async_copy(f8_hbm.at[stage.at[i]], rbuf.at[i], sem)

    def drain_g(rbuf, sem):
        for i in range(_K):
            pltpu.make_async_copy(f8_hbm.at[srcv.at[0]], rbuf.at[i], sem).wait()

    def fire_s(t, rbuf, sem):
        for i in range(_K):
            pltpu.async_copy(rbuf.at[i], acc.at[dstv.at[t * _K + i]], sem,
                             add=True)

    def drain_s(t, rbuf, sem):
        for i in range(_K):
            pltpu.make_async_copy(rbuf.at[i], acc.at[dstv.at[t * _K + i]],
                                  sem).wait()

    @pl.loop(0, _NPASS)
    def _(p):
        bvec = jnp.broadcast_to(_NPASS * c + p, (16,)).astype(jnp.int32)

        pltpu.sync_copy(z_hbm, acc.at[pl.ds(base, _NPS)])

        plsc.subcore_barrier()

        fire_g(0, stage_a, rows_a, gs_a, bvec)

        @pl.loop(0, _T // 2)
        def _(u):
            ta = 2 * u
            tb = 2 * u + 1
            fire_g(tb, stage_b, rows_b, gs_b, bvec)
            drain_g(rows_a, gs_a)
            fire_s(ta, rows_a, ss_a)
            drain_g(rows_b, gs_b)
            fire_s(tb, rows_b, ss_b)
            drain_s(ta, rows_a, ss_a)

            @pl.when(u + 1 < _T // 2)
            def _():
                fire_g(ta + 2, stage_a, rows_a, gs_a, bvec)

            drain_s(tb, rows_b, ss_b)

        plsc.subcore_barrier()

        col = _BD * (_NPASS * c + p)

        @pl.when(s == _NS - 1)
        def _():
            pltpu.sync_copy(acc.at[pl.ds(base, _LAST)],
                            out_hbm.at[pl.ds(base, _LAST), pl.ds(col, _BD)])

        @pl.when(s != _NS - 1)
        def _():
            pltpu.sync_copy(acc.at[pl.ds(base, _NPS)],
                            out_hbm.at[pl.ds(base, _NPS), pl.ds(col, _BD)])

        plsc.subcore_barrier()


@jax.jit
def _sc_aggregate(f8, ei_r, z):
    mesh = plsc.VectorSubcoreMesh(core_axis_name="c", subcore_axis_name="s")
    run = pl.kernel(
        _sc_body,
        out_type=jax.ShapeDtypeStruct((_N, _D), jnp.float32),
        mesh=mesh,
        compiler_params=pltpu.CompilerParams(use_tc_tiling_on_sc=False),
        scratch_types=[
            pltpu.VMEM((_CH, _W), jnp.int32),
            pltpu.VMEM((_CH, _W), jnp.int32),
            pltpu.VMEM((_K, _W), jnp.int32),
            pltpu.VMEM((_K, _W), jnp.int32),
            pltpu.VMEM((_K, _W, _BD), jnp.float32),
            pltpu.VMEM((_K, _W, _BD), jnp.float32),
            pltpu.VMEM_SHARED((_NP, _BD), jnp.float32),
            pltpu.SemaphoreType.DMA,
            pltpu.SemaphoreType.DMA,
            pltpu.SemaphoreType.DMA,
            pltpu.SemaphoreType.DMA,
        ],
    )
    return run(f8, ei_r, z)


def kernel(features, edge_index):
    ei = edge_index.astype(jnp.int32)
    pad = _EPAD - _E
    ei_pad = jnp.concatenate(
        [ei, jnp.stack([jnp.zeros((pad,), jnp.int32),
                        jnp.full((pad,), _NP - 1, jnp.int32)])], axis=1)
    ei_r = ei_pad.reshape(2, _NS, _CH, _W)
    f8 = features.reshape(_N * 8, _BD)
    z = jnp.zeros((_NPS, _BD), jnp.float32)
    return _sc_aggregate(f8, ei_r, z)


# speedup vs baseline: 1.5971x; 1.5971x over previous
import functools

import jax
import jax.numpy as jnp
from jax import lax
from jax.experimental import pallas as pl
from jax.experimental.pallas import tpu as pltpu
from jax.experimental.pallas import tpu_sc as plsc

_N = 50000
_E = 800000
_D = 64
_NC = 2
_NS = 16
_NPASS = 2
_W = 128
_K = 2
_CH = 392
_EPAD = _NS * _W * _CH
_NPS = 3128
_NP = _NS * _NPS
_BD = _D // (_NC * _NPASS)
_LAST = _N - (_NS - 1) * _NPS
_T = _CH // _K


def _sc_body(fb_hbm, ei_hbm, z_hbm, out_hbm, srcv, dstv, rows_a, rows_b,
             acc, gs_a, gs_b, ss_a, ss_b):
    c = lax.axis_index("c")
    s = lax.axis_index("s")
    base = s * _NPS

    pltpu.sync_copy(ei_hbm.at[0, s], srcv)
    pltpu.sync_copy(ei_hbm.at[1, s], dstv)

    def fire_g(t, rbuf, sem):
        for i in range(_K):
            pltpu.async_copy(fb_hbm.at[srcv.at[t * _K + i]], rbuf.at[i], sem)

    def drain_g(rbuf, sem):
        for i in range(_K):
            pltpu.make_async_copy(fb_hbm.at[srcv.at[0]], rbuf.at[i], sem).wait()

    def fire_s(t, rbuf, sem):
        for i in range(_K):
            pltpu.async_copy(rbuf.at[i], acc.at[dstv.at[t * _K + i]], sem,
                             add=True)

    def drain_s(t, rbuf, sem):
        for i in range(_K):
            pltpu.make_async_copy(rbuf.at[i], acc.at[dstv.at[t * _K + i]],
                                  sem).wait()

    @pl.loop(0, _NPASS)
    def _(p):
        pltpu.sync_copy(z_hbm, acc.at[pl.ds(base, _NPS)])

        @pl.when(p == 0)
        def _():
            bvec = jnp.broadcast_to(_NPASS * c, (16,)).astype(jnp.int32)

            @pl.loop(0, _CH)
            def _(j):
                for k in range(8):
                    sl = (j, pl.ds(k * 16, 16))
                    srcv[sl] = srcv[sl] * 4 + bvec

        @pl.when(p != 0)
        def _():
            one = jnp.ones((16,), jnp.int32)

            @pl.loop(0, _CH)
            def _(j):
                for k in range(8):
                    sl = (j, pl.ds(k * 16, 16))
                    srcv[sl] = srcv[sl] + one

        plsc.subcore_barrier()

        fire_g(0, rows_a, gs_a)

        @pl.loop(0, _T // 2)
        def _(u):
            ta = 2 * u
            tb = 2 * u + 1
            fire_g(tb, rows_b, gs_b)
            drain_g(rows_a, gs_a)
            fire_s(ta, rows_a, ss_a)
            drain_g(rows_b, gs_b)
            fire_s(tb, rows_b, ss_b)
            drain_s(ta, rows_a, ss_a)

            @pl.when(u + 1 < _T // 2)
            def _():
                fire_g(ta + 2, rows_a, gs_a)

            drain_s(tb, rows_b, ss_b)

        plsc.subcore_barrier()

        col = _BD * (_NPASS * c + p)

        @pl.when(s == _NS - 1)
        def _():
            pltpu.sync_copy(acc.at[pl.ds(base, _LAST)],
                            out_hbm.at[pl.ds(base, _LAST), pl.ds(col, _BD)])

        @pl.when(s != _NS - 1)
        def _():
            pltpu.sync_copy(acc.at[pl.ds(base, _NPS)],
                            out_hbm.at[pl.ds(base, _NPS), pl.ds(col, _BD)])

        plsc.subcore_barrier()


@jax.jit
def _sc_aggregate_bf(fb, ei_r, z):
    mesh = plsc.VectorSubcoreMesh(core_axis_name="c", subcore_axis_name="s")
    run = pl.kernel(
        _sc_body,
        out_type=jax.ShapeDtypeStruct((_N, _D), jnp.bfloat16),
        mesh=mesh,
        compiler_params=pltpu.CompilerParams(use_tc_tiling_on_sc=False),
        scratch_types=[
            pltpu.VMEM((_CH, _W), jnp.int32),
            pltpu.VMEM((_CH, _W), jnp.int32),
            pltpu.VMEM((_K, _W, _BD), jnp.bfloat16),
            pltpu.VMEM((_K, _W, _BD), jnp.bfloat16),
            pltpu.VMEM_SHARED((_NP, _BD), jnp.bfloat16),
            pltpu.SemaphoreType.DMA,
            pltpu.SemaphoreType.DMA,
            pltpu.SemaphoreType.DMA,
            pltpu.SemaphoreType.DMA,
        ],
    )
    return run(fb, ei_r, z)


def kernel(features, edge_index):
    ei = edge_index.astype(jnp.int32)
    pad = _EPAD - _E
    ei_pad = jnp.concatenate(
        [ei, jnp.stack([jnp.zeros((pad,), jnp.int32),
                        jnp.full((pad,), _NP - 1, jnp.int32)])], axis=1)
    ei_r = ei_pad.reshape(2, _NS, _CH, _W)
    fb = features.astype(jnp.bfloat16).reshape(_N * 4, _BD)
    z = jnp.zeros((_NPS, _BD), jnp.bfloat16)
    return _sc_aggregate_bf(fb, ei_r, z).astype(jnp.float32)


# speedup vs baseline: 1.6038x; 1.0042x over previous
import functools

import jax
import jax.numpy as jnp
from jax import lax
from jax.experimental import pallas as pl
from jax.experimental.pallas import tpu as pltpu
from jax.experimental.pallas import tpu_sc as plsc

_N = 50000
_E = 800000
_D = 64
_NC = 2
_NS = 16
_NPASS = 2
_W = 128
_K = 2
_CH = 392
_EPAD = _NS * _W * _CH
_NPS = 3128
_NP = _NS * _NPS
_BD = _D // (_NC * _NPASS)
_LAST = _N - (_NS - 1) * _NPS
_T = _CH // _K


def _sc_body(fb_hbm, ei_hbm, z_hbm, out_hbm, srcv, dstv, rows_a, rows_b,
             acc, gs_a, gs_b, ss_a, ss_b):
    c = lax.axis_index("c")
    s = lax.axis_index("s")
    base = s * _NPS

    pltpu.sync_copy(ei_hbm.at[0, s], srcv)
    pltpu.sync_copy(ei_hbm.at[1, s], dstv)

    def fire_g(t, rbuf, sem):
        for i in range(_K):
            pltpu.async_copy(fb_hbm.at[srcv.at[t * _K + i]], rbuf.at[i], sem)

    def drain_g(rbuf, sem):
        for i in range(_K):
            pltpu.make_async_copy(fb_hbm.at[srcv.at[0]], rbuf.at[i], sem).wait()

    def fire_s(t, rbuf, sem):
        for i in range(_K):
            pltpu.async_copy(rbuf.at[i], acc.at[dstv.at[t * _K + i]], sem,
                             add=True)

    def drain_s(t, rbuf, sem):
        for i in range(_K):
            pltpu.make_async_copy(rbuf.at[i], acc.at[dstv.at[t * _K + i]],
                                  sem).wait()

    @pl.loop(0, _NPASS)
    def _(p):
        pltpu.sync_copy(z_hbm, acc.at[pl.ds(base, _NPS)])

        @pl.when(p == 0)
        def _():
            bvec = jnp.broadcast_to(_NPASS * c, (16,)).astype(jnp.int32)

            @pl.loop(0, _CH)
            def _(j):
                for k in range(8):
                    sl = (j, pl.ds(k * 16, 16))
                    srcv[sl] = srcv[sl] * 4 + bvec

        @pl.when(p != 0)
        def _():
            one = jnp.ones((16,), jnp.int32)

            @pl.loop(0, _CH)
            def _(j):
                for k in range(8):
                    sl = (j, pl.ds(k * 16, 16))
                    srcv[sl] = srcv[sl] + one

        plsc.subcore_barrier()

        fire_g(0, rows_a, gs_a)
        fire_g(1, rows_b, gs_b)

        @pl.loop(0, _T // 2)
        def _(u):
            ta = 2 * u
            tb = 2 * u + 1
            drain_g(rows_a, gs_a)
            fire_s(ta, rows_a, ss_a)
            drain_g(rows_b, gs_b)
            fire_s(tb, rows_b, ss_b)
            drain_s(ta, rows_a, ss_a)

            @pl.when(u + 1 < _T // 2)
            def _():
                fire_g(ta + 2, rows_a, gs_a)

            drain_s(tb, rows_b, ss_b)

            @pl.when(u + 1 < _T // 2)
            def _():
                fire_g(tb + 2, rows_b, gs_b)

        plsc.subcore_barrier()

        col = _BD * (_NPASS * c + p)

        @pl.when(s == _NS - 1)
        def _():
            pltpu.sync_copy(acc.at[pl.ds(base, _LAST)],
                            out_hbm.at[pl.ds(base, _LAST), pl.ds(col, _BD)])

        @pl.when(s != _NS - 1)
        def _():
            pltpu.sync_copy(acc.at[pl.ds(base, _NPS)],
                            out_hbm.at[pl.ds(base, _NPS), pl.ds(col, _BD)])

        plsc.subcore_barrier()


@jax.jit
def _sc_aggregate_bf(fb, ei_r, z):
    mesh = plsc.VectorSubcoreMesh(core_axis_name="c", subcore_axis_name="s")
    run = pl.kernel(
        _sc_body,
        out_type=jax.ShapeDtypeStruct((_N, _D), jnp.bfloat16),
        mesh=mesh,
        compiler_params=pltpu.CompilerParams(use_tc_tiling_on_sc=False),
        scratch_types=[
            pltpu.VMEM((_CH, _W), jnp.int32),
            pltpu.VMEM((_CH, _W), jnp.int32),
            pltpu.VMEM((_K, _W, _BD), jnp.bfloat16),
            pltpu.VMEM((_K, _W, _BD), jnp.bfloat16),
            pltpu.VMEM_SHARED((_NP, _BD), jnp.bfloat16),
            pltpu.SemaphoreType.DMA,
            pltpu.SemaphoreType.DMA,
            pltpu.SemaphoreType.DMA,
            pltpu.SemaphoreType.DMA,
        ],
    )
    return run(fb, ei_r, z)


def kernel(features, edge_index):
    ei = edge_index.astype(jnp.int32)
    pad = _EPAD - _E
    ei_pad = jnp.concatenate(
        [ei, jnp.stack([jnp.zeros((pad,), jnp.int32),
                        jnp.full((pad,), _NP - 1, jnp.int32)])], axis=1)
    ei_r = ei_pad.reshape(2, _NS, _CH, _W)
    fb = features.astype(jnp.bfloat16).reshape(_N * 4, _BD)
    z = jnp.zeros((_NPS, _BD), jnp.bfloat16)
    return _sc_aggregate_bf(fb, ei_r, z).astype(jnp.float32)


# speedup vs baseline: 1.6046x; 1.0005x over previous
import functools

import jax
import jax.numpy as jnp
from jax import lax
from jax.experimental import pallas as pl
from jax.experimental.pallas import tpu as pltpu
from jax.experimental.pallas import tpu_sc as plsc

_N = 50000
_E = 800000
_D = 64
_NC = 2
_NS = 16
_NPASS = 2
_W = 128
_K = 2
_CH = 392
_EPAD = _NS * _W * _CH
_NPS = 3128
_NP = _NS * _NPS
_BD = _D // (_NC * _NPASS)
_LAST = _N - (_NS - 1) * _NPS
_T = _CH // _K


def _sc_body(fb_hbm, ei_hbm, z_hbm, out_hbm, srcv, dstv, rows_a, rows_b,
             acc, gs_a, gs_b, ss_a, ss_b):
    c = lax.axis_index("c")
    s = lax.axis_index("s")
    base = s * _NPS

    pltpu.sync_copy(ei_hbm.at[0, s], srcv)
    pltpu.sync_copy(ei_hbm.at[1, s], dstv)

    def fire_g(t, rbuf, sem):
        for i in range(_K):
            pltpu.async_copy(fb_hbm.at[srcv.at[t * _K + i]], rbuf.at[i], sem)

    def drain_g(rbuf, sem):
        for i in range(_K):
            pltpu.make_async_copy(fb_hbm.at[srcv.at[0]], rbuf.at[i], sem).wait()

    def fire_s(t, rbuf, sem):
        for i in range(_K):
            pltpu.async_copy(rbuf.at[i], acc.at[dstv.at[t * _K + i]], sem,
                             add=True)

    def drain_s(t, rbuf, sem):
        for i in range(_K):
            pltpu.make_async_copy(rbuf.at[i], acc.at[dstv.at[t * _K + i]],
                                  sem).wait()

    @pl.loop(0, _NPASS)
    def _(p):
        pltpu.sync_copy(z_hbm, acc.at[pl.ds(base, _NPS)])

        @pl.when(p == 0)
        def _():
            bvec = jnp.broadcast_to(_NPASS * c, (16,)).astype(jnp.int32)

            @pl.loop(0, _CH)
            def _(j):
                for k in range(8):
                    sl = (j, pl.ds(k * 16, 16))
                    srcv[sl] = srcv[sl] * 4 + bvec

        @pl.when(p != 0)
        def _():
            one = jnp.ones((16,), jnp.int32)

            @pl.loop(0, _CH)
            def _(j):
                for k in range(8):
                    sl = (j, pl.ds(k * 16, 16))
                    srcv[sl] = srcv[sl] + one

        plsc.subcore_barrier()

        fire_g(0, rows_a, gs_a)

        @pl.loop(0, _T // 2)
        def _(u):
            ta = 2 * u
            tb = 2 * u + 1
            fire_g(tb, rows_b, gs_b)
            drain_g(rows_a, gs_a)
            fire_s(ta, rows_a, ss_a)
            drain_g(rows_b, gs_b)
            fire_s(tb, rows_b, ss_b)
            drain_s(ta, rows_a, ss_a)

            @pl.when(u + 1 < _T // 2)
            def _():
                fire_g(ta + 2, rows_a, gs_a)

            drain_s(tb, rows_b, ss_b)

        plsc.subcore_barrier()

        col = _BD * (_NPASS * c + p)

        @pl.when(s == _NS - 1)
        def _():
            pltpu.sync_copy(acc.at[pl.ds(base, _LAST)],
                            out_hbm.at[pl.ds(base, _LAST), pl.ds(col, _BD)])

        @pl.when(s != _NS - 1)
        def _():
            pltpu.sync_copy(acc.at[pl.ds(base, _NPS)],
                            out_hbm.at[pl.ds(base, _NPS), pl.ds(col, _BD)])

        plsc.subcore_barrier()


@jax.jit
def _sc_aggregate_bf(fb, ei_r, z):
    mesh = plsc.VectorSubcoreMesh(core_axis_name="c", subcore_axis_name="s")
    run = pl.kernel(
        _sc_body,
        out_type=jax.ShapeDtypeStruct((_N, _D), jnp.bfloat16),
        mesh=mesh,
        compiler_params=pltpu.CompilerParams(use_tc_tiling_on_sc=False),
        scratch_types=[
            pltpu.VMEM((_CH, _W), jnp.int32),
            pltpu.VMEM((_CH, _W), jnp.int32),
            pltpu.VMEM((_K, _W, _BD), jnp.bfloat16),
            pltpu.VMEM((_K, _W, _BD), jnp.bfloat16),
            pltpu.VMEM_SHARED((_NP, _BD), jnp.bfloat16),
            pltpu.SemaphoreType.DMA,
            pltpu.SemaphoreType.DMA,
            pltpu.SemaphoreType.DMA,
            pltpu.SemaphoreType.DMA,
        ],
    )
    return run(fb, ei_r, z)


def kernel(features, edge_index):
    ei = edge_index.astype(jnp.int32)
    pad = _EPAD - _E
    ei_pad = jnp.concatenate(
        [ei, jnp.stack([jnp.zeros((pad,), jnp.int32),
                        jnp.full((pad,), _NP - 1, jnp.int32)])], axis=1)
    ei_r = ei_pad.reshape(2, _NS, _CH, _W)
    fb = features.astype(jnp.bfloat16).reshape(_N * 4, _BD)
    z = jnp.zeros((_NPS, _BD), jnp.bfloat16)
    return _sc_aggregate_bf(fb, ei_r, z).astype(jnp.float32)
